# baseline (speedup 1.0000x reference)
import functools

import jax
import jax.numpy as jnp
from jax import lax
from jax.experimental import pallas as pl
from jax.experimental.pallas import tpu as pltpu
from jax.experimental.pallas import tpu_sc as plsc

_N = 7056
_NW = 16
_CHUNK = 448
_VECS = _CHUNK // 16
_TAIL = _N - (_NW - 1) * _CHUNK
_TVECS = _TAIL // 16

_mesh = plsc.VectorSubcoreMesh(
    core_axis_name="c", subcore_axis_name="s", num_cores=1, num_subcores=16
)


def _bfly(v, op):
    iota = lax.iota(jnp.int32, 16)
    for k in (8, 4, 2, 1):
        v = op(v, v.at[iota ^ k].get(mode="promise_in_bounds"))
    return v


@functools.partial(
    pl.kernel,
    out_type=jax.ShapeDtypeStruct((_N,), jnp.float32),
    mesh=_mesh,
    scratch_types=[
        pltpu.VMEM((_CHUNK,), jnp.float32),
        pltpu.VMEM((_CHUNK,), jnp.float32),
        pltpu.VMEM((2, 16), jnp.float32),
        pltpu.VMEM((16, 2, 16), jnp.float32),
        pltpu.VMEM((16,), jnp.float32),
        pltpu.HBM((16, 2, 16), jnp.float32),
        pltpu.SemaphoreType.DMA,
        pltpu.SemaphoreType.DMA,
    ],
)
def _sc_argmax_onehot(x_hbm, o_hbm, xbuf, zbuf, cbuf, msbuf, ohbuf, sh,
                      sem_in, sem_z):
    s = lax.axis_index("s")
    iota = lax.iota(jnp.int32, 16)
    last = s == _NW - 1
    base = s * _CHUNK

    @pl.when(jnp.logical_not(last))
    def _load_full():
        pltpu.async_copy(x_hbm.at[pl.ds(base, _CHUNK)], xbuf, sem_in)

    @pl.when(last)
    def _load_tail():
        pltpu.async_copy(x_hbm.at[pl.ds(base, _TAIL)],
                         xbuf.at[pl.ds(0, _TAIL)], sem_in)

    zeros = jnp.zeros((16,), jnp.float32)
    for j in range(_VECS):
        zbuf[pl.ds(j * 16, 16)] = zeros

    @pl.when(jnp.logical_not(last))
    def _zero_full():
        pltpu.async_copy(zbuf, o_hbm.at[pl.ds(base, _CHUNK)], sem_z)

    @pl.when(last)
    def _zero_tail():
        pltpu.async_copy(zbuf.at[pl.ds(0, _TAIL)],
                         o_hbm.at[pl.ds(base, _TAIL)], sem_z)

    @pl.when(jnp.logical_not(last))
    def _wait_full():
        pltpu.make_async_copy(x_hbm.at[pl.ds(base, _CHUNK)], xbuf,
                              sem_in).wait()

    @pl.when(last)
    def _wait_tail():
        pltpu.make_async_copy(x_hbm.at[pl.ds(base, _TAIL)],
                              xbuf.at[pl.ds(0, _TAIL)], sem_in).wait()

    m = xbuf[pl.ds(0, 16)]
    mi = iota + base
    for j in range(1, _VECS):
        v = xbuf[pl.ds(j * 16, 16)]
        iv = iota + (base + j * 16)
        upd = v > m
        if j >= _TVECS:
            upd = jnp.logical_and(upd, iv < _N)
        m = jnp.where(upd, v, m)
        mi = jnp.where(upd, iv, mi)

    wmaxv = _bfly(m, jnp.maximum)
    widxv = _bfly(jnp.where(m == wmaxv, mi, _N), jnp.minimum)
    cbuf[0] = wmaxv
    cbuf[1] = widxv.astype(jnp.float32)
    pltpu.sync_copy(cbuf, sh.at[s])

    @pl.when(jnp.logical_not(last))
    def _wait_zero_full():
        pltpu.make_async_copy(zbuf, o_hbm.at[pl.ds(base, _CHUNK)],
                              sem_z).wait()

    @pl.when(last)
    def _wait_zero_tail():
        pltpu.make_async_copy(zbuf.at[pl.ds(0, _TAIL)],
                              o_hbm.at[pl.ds(base, _TAIL)], sem_z).wait()

    plsc.subcore_barrier()

    @pl.when(s == 0)
    def _merge():
        pltpu.sync_copy(sh, msbuf)
        gv = msbuf[0, 0]
        for r in range(1, _NW):
            gv = jnp.maximum(gv, msbuf[r, 0])
        givf = jnp.where(msbuf[0, 0] == gv, msbuf[0, 1], float(_N))
        for r in range(1, _NW):
            givf = jnp.minimum(
                givf, jnp.where(msbuf[r, 0] == gv, msbuf[r, 1], float(_N)))
        giv = givf.astype(jnp.int32)
        ohbuf[...] = jnp.where(iota == (giv & 15), 1.0,
                               0.0).astype(jnp.float32)
        win = (giv[0] // 16) * 16
        pltpu.sync_copy(ohbuf, o_hbm.at[pl.ds(win, 16)])


@jax.jit
def kernel(inputs):
    flat = jnp.reshape(inputs, (_N,))
    return jnp.reshape(_sc_argmax_onehot(flat), (1, _N))


# speedup vs baseline: 1.0010x; 1.0010x over previous
import functools

import jax
import jax.numpy as jnp
from jax import lax
from jax.experimental import pallas as pl
from jax.experimental.pallas import tpu as pltpu
from jax.experimental.pallas import tpu_sc as plsc

_N = 7056
_NW = 16
_G = 4
_GW = _G * 84
_CHUNK = 448
_TAIL = _N - (_NW - 1) * _CHUNK
_COLS = (0, 16, 32, 48, 64, 68)

_mesh = plsc.VectorSubcoreMesh(
    core_axis_name="c", subcore_axis_name="s", num_cores=1, num_subcores=16
)


def _bfly(v, op):
    iota = lax.iota(jnp.int32, 16)
    for k in (8, 4, 2, 1):
        v = op(v, v.at[iota ^ k].get(mode="promise_in_bounds"))
    return v


@functools.partial(
    pl.kernel,
    out_type=jax.ShapeDtypeStruct((1, _N), jnp.float32),
    mesh=_mesh,
    compiler_params=pltpu.CompilerParams(use_tc_tiling_on_sc=False),
    scratch_types=[
        pltpu.VMEM((2 * _G, 84), jnp.float32),
        pltpu.VMEM((_CHUNK,), jnp.float32),
        pltpu.VMEM((2, 16), jnp.float32),
        pltpu.VMEM((16, 2, 16), jnp.float32),
        pltpu.VMEM((16,), jnp.float32),
        pltpu.HBM((16, 2, 16), jnp.float32),
        pltpu.SemaphoreType.DMA,
        pltpu.SemaphoreType.DMA,
    ],
)
def _sc_argmax_onehot(x_hbm, o_hbm, xbuf, zbuf, cbuf, msbuf, ohbuf, sh,
                      sem_in, sem_z):
    s = lax.axis_index("s")
    iota = lax.iota(jnp.int32, 16)
    last = s == _NW - 1
    gbase = s * _GW
    zbase = s * _CHUNK

    in_a = pltpu.make_async_copy(
        x_hbm.at[pl.ds(s * _G, _G)], xbuf.at[pl.ds(0, _G)], sem_in)
    in_b = pltpu.make_async_copy(
        x_hbm.at[pl.ds((16 + s) * _G, _G)], xbuf.at[pl.ds(_G, _G)], sem_in)
    in_a.start()

    @pl.when(s < 5)
    def _load_b():
        in_b.start()

    zeros = jnp.zeros((16,), jnp.float32)
    for j in range(_CHUNK // 16):
        zbuf[pl.ds(j * 16, 16)] = zeros

    z_full = pltpu.make_async_copy(
        zbuf, o_hbm.at[0, pl.ds(zbase, _CHUNK)], sem_z)
    z_tail = pltpu.make_async_copy(
        zbuf.at[pl.ds(0, _TAIL)], o_hbm.at[0, pl.ds(zbase, _TAIL)], sem_z)

    @pl.when(jnp.logical_not(last))
    def _zero_full():
        z_full.start()

    @pl.when(last)
    def _zero_tail():
        z_tail.start()

    in_a.wait()

    @pl.when(s < 5)
    def _wait_b():
        in_b.wait()

    m = xbuf[0, pl.ds(0, 16)]
    mi = iota + gbase
    first = True
    for r in range(2 * _G):
        roff = (gbase + 84 * r) if r < _G else (5376 + gbase + 84 * (r - _G))
        for c in _COLS:
            if first:
                first = False
                continue
            v = xbuf[r, pl.ds(c, 16)]
            iv = iota + (roff + c)
            upd = v > m
            if r >= _G:
                upd = jnp.logical_and(upd, iv < _N)
            m = jnp.where(upd, v, m)
            mi = jnp.where(upd, iv, mi)

    wmaxv = _bfly(m, jnp.maximum)
    widxv = _bfly(jnp.where(m == wmaxv, mi, _N), jnp.minimum)
    cbuf[0] = wmaxv
    cbuf[1] = widxv.astype(jnp.float32)
    pltpu.sync_copy(cbuf, sh.at[s])

    @pl.when(jnp.logical_not(last))
    def _wait_zero_full():
        z_full.wait()

    @pl.when(last)
    def _wait_zero_tail():
        z_tail.wait()

    plsc.subcore_barrier()

    @pl.when(s == 0)
    def _merge():
        pltpu.sync_copy(sh, msbuf)
        gv = msbuf[0, 0]
        for r in range(1, _NW):
            gv = jnp.maximum(gv, msbuf[r, 0])
        givf = jnp.where(msbuf[0, 0] == gv, msbuf[0, 1], float(_N))
        for r in range(1, _NW):
            givf = jnp.minimum(
                givf, jnp.where(msbuf[r, 0] == gv, msbuf[r, 1], float(_N)))
        giv = givf.astype(jnp.int32)
        ohbuf[...] = jnp.where(iota == (giv & 15), 1.0,
                               0.0).astype(jnp.float32)
        win = (giv[0] // 16) * 16
        pltpu.sync_copy(ohbuf, o_hbm.at[0, pl.ds(win, 16)])


@jax.jit
def kernel(inputs):
    return _sc_argmax_onehot(inputs)


# speedup vs baseline: 1.0091x; 1.0080x over previous
import functools

import jax
import jax.numpy as jnp
from jax import lax
from jax.experimental import pallas as pl
from jax.experimental.pallas import tpu as pltpu
from jax.experimental.pallas import tpu_sc as plsc

_N = 7056
_NW = 16
_G = 4
_GW = _G * 84
_CHUNK = 448
_TAIL = _N - (_NW - 1) * _CHUNK
_COLS = (0, 16, 32, 48, 64, 68)

_mesh = plsc.VectorSubcoreMesh(
    core_axis_name="c", subcore_axis_name="s", num_cores=1, num_subcores=16
)


def _bfly(v, op):
    iota = lax.iota(jnp.int32, 16)
    for k in (8, 4, 2, 1):
        v = op(v, v.at[iota ^ k].get(mode="promise_in_bounds"))
    return v


@functools.partial(
    pl.kernel,
    out_type=jax.ShapeDtypeStruct((1, _N), jnp.float32),
    mesh=_mesh,
    compiler_params=pltpu.CompilerParams(use_tc_tiling_on_sc=False),
    scratch_types=[
        pltpu.VMEM((2 * _G, 84), jnp.float32),
        pltpu.VMEM((_CHUNK,), jnp.float32),
        pltpu.VMEM((2, 16), jnp.float32),
        pltpu.VMEM((16, 2, 16), jnp.float32),
        pltpu.VMEM((16,), jnp.float32),
        pltpu.HBM((16, 2, 16), jnp.float32),
        pltpu.SemaphoreType.DMA,
        pltpu.SemaphoreType.DMA,
        pltpu.SemaphoreType.DMA,
    ],
)
def _sc_argmax_onehot(x_hbm, o_hbm, xbuf, zbuf, cbuf, msbuf, ohbuf, sh,
                      sem_in, sem_z, sem_p):
    s = lax.axis_index("s")
    iota = lax.iota(jnp.int32, 16)
    last = s == _NW - 1
    gbase = s * _GW
    zbase = s * _CHUNK

    in_a = pltpu.make_async_copy(
        x_hbm.at[pl.ds(s * _G, _G)], xbuf.at[pl.ds(0, _G)], sem_in)
    in_b = pltpu.make_async_copy(
        x_hbm.at[pl.ds((16 + s) * _G, _G)], xbuf.at[pl.ds(_G, _G)], sem_in)
    in_a.start()

    @pl.when(s < 5)
    def _load_b():
        in_b.start()

    zeros = jnp.zeros((16,), jnp.float32)

    def _zbody(j, carry):
        zbuf[pl.ds(j * 16, 16)] = zeros
        return carry

    lax.fori_loop(0, _CHUNK // 16, _zbody, 0)

    z_full = pltpu.make_async_copy(
        zbuf, o_hbm.at[0, pl.ds(zbase, _CHUNK)], sem_z)
    z_tail = pltpu.make_async_copy(
        zbuf.at[pl.ds(0, _TAIL)], o_hbm.at[0, pl.ds(zbase, _TAIL)], sem_z)

    @pl.when(jnp.logical_not(last))
    def _zero_full():
        z_full.start()

    @pl.when(last)
    def _zero_tail():
        z_tail.start()

    in_a.wait()

    @pl.when(s < 5)
    def _wait_b():
        in_b.wait()

    m0 = jnp.full((16,), -jnp.inf, jnp.float32)
    mi0 = jnp.zeros((16,), jnp.int32)

    def _sbody(r, carry):
        m, mi = carry
        roff = gbase + 84 * r + jnp.where(r >= _G, 5376 - _GW, 0)
        for c in _COLS:
            v = xbuf[r, pl.ds(c, 16)]
            iv = iota + (roff + c)
            upd = jnp.logical_and(v > m, iv < _N)
            m = jnp.where(upd, v, m)
            mi = jnp.where(upd, iv, mi)
        return m, mi

    m, mi = lax.fori_loop(0, 2 * _G, _sbody, (m0, mi0))

    wmaxv = _bfly(m, jnp.maximum)
    widxv = _bfly(jnp.where(m == wmaxv, mi, _N), jnp.minimum)
    cbuf[0] = wmaxv
    cbuf[1] = widxv.astype(jnp.float32)
    pub = pltpu.make_async_copy(cbuf, sh.at[s], sem_p)
    pub.start()

    @pl.when(jnp.logical_not(last))
    def _wait_zero_full():
        z_full.wait()

    @pl.when(last)
    def _wait_zero_tail():
        z_tail.wait()

    pub.wait()
    plsc.subcore_barrier()

    @pl.when(s == 0)
    def _merge():
        pltpu.sync_copy(sh, msbuf)
        gv = msbuf[0, 0]
        for r in range(1, _NW):
            gv = jnp.maximum(gv, msbuf[r, 0])
        givf = jnp.where(msbuf[0, 0] == gv, msbuf[0, 1], float(_N))
        for r in range(1, _NW):
            givf = jnp.minimum(
                givf, jnp.where(msbuf[r, 0] == gv, msbuf[r, 1], float(_N)))
        giv = givf.astype(jnp.int32)
        ohbuf[...] = jnp.where(iota == (giv & 15), 1.0,
                               0.0).astype(jnp.float32)
        win = (giv[0] // 16) * 16
        pltpu.sync_copy(ohbuf, o_hbm.at[0, pl.ds(win, 16)])


@jax.jit
def kernel(inputs):
    return _sc_argmax_onehot(inputs)
